# baseline (speedup 1.0000x reference)
import jax
import jax.numpy as jnp
from jax.experimental import pallas as pl
from jax.experimental.pallas import tpu as pltpu

_N = 2048
_R = 1024
_NB = _N // _R


def _layer_body(x_ref, w_ref, b_ref, mw_ref, mb_ref, out_w_ref, act_ref,
                sems):
    pid = pl.program_id(0)
    w = w_ref[...]
    x = x_ref[...]
    y = jnp.sum(w * x, axis=1)
    a = jnp.maximum(y + b_ref[...], 0.0)
    act_ref[...] = a[None, :]

    @pl.when(pid == 0)
    def _():
        a0 = jnp.sum(jnp.where(jax.lax.iota(jnp.int32, _R) == 0, a, 0.0))
        patch = (mw_ref[0, 0] * x_ref[0, 0]
                 + mw_ref[0, 1] * w_ref[0, 0]
                 + mw_ref[0, 2] * a0
                 + mb_ref[0])
        rows = jax.lax.broadcasted_iota(jnp.int32, (8, 128), 0)
        cols = jax.lax.broadcasted_iota(jnp.int32, (8, 128), 1)
        tile = w_ref[0:8, 0:128]
        w_ref[0:8, 0:128] = jnp.where((rows == 0) & (cols == 0), patch, tile)

    @pl.when(pid > 0)
    def _():
        pltpu.make_async_copy(
            w_ref, out_w_ref.at[pl.ds((pid - 1) * _R, _R), :],
            sems.at[(pid - 1) % 2]).wait()

    copy = pltpu.make_async_copy(
        w_ref, out_w_ref.at[pl.ds(pid * _R, _R), :], sems.at[pid % 2])
    copy.start()

    @pl.when(pid == _NB - 1)
    def _():
        copy.wait()


def _layer(x, w, b, mw, mb, interpret=False):
    return pl.pallas_call(
        _layer_body,
        grid=(_NB,),
        in_specs=[
            pl.BlockSpec((1, _N), lambda i: (0, 0)),
            pl.BlockSpec((_R, _N), lambda i: (i, 0)),
            pl.BlockSpec((_R,), lambda i: (i,)),
            pl.BlockSpec((1, 3), lambda i: (0, 0)),
            pl.BlockSpec((1,), lambda i: (0,)),
        ],
        out_specs=[
            pl.BlockSpec(memory_space=pl.ANY),
            pl.BlockSpec((1, _R), lambda i: (0, i)),
        ],
        out_shape=[
            jax.ShapeDtypeStruct((_N, _N), jnp.float32),
            jax.ShapeDtypeStruct((1, _N), jnp.float32),
        ],
        scratch_shapes=[pltpu.SemaphoreType.DMA((2,))],
        interpret=interpret,
    )(x, w, b, mw, mb)


def kernel(x, W1, b1, W2, b2, W3, b3, meta_W, meta_b):
    nw1, h1 = _layer(x, W1, b1, meta_W, meta_b)
    nw2, h2 = _layer(h1, W2, b2, meta_W, meta_b)
    nw3, out = _layer(h2, W3, b3, meta_W, meta_b)
    return out, nw1, nw2, nw3


# speedup vs baseline: 1.2817x; 1.2817x over previous
import jax
import jax.numpy as jnp
from jax.experimental import pallas as pl
from jax.experimental.pallas import tpu as pltpu

_N = 2048
_R = 512
_NB = _N // _R
_M = 3 * _NB
_NBUF = 4
_D = 2


def _body(x_ref, b1_ref, b2_ref, b3_ref, mw_ref, mb_ref,
          w1_hbm, w2_hbm, w3_hbm,
          o1_hbm, o2_hbm, o3_hbm, out_ref,
          bufs, act1, act2, load_sems, store_sems):
    w_hbms = [w1_hbm, w2_hbm, w3_hbm]
    o_hbms = [o1_hbm, o2_hbm, o3_hbm]
    b_refs = [b1_ref, b2_ref, b3_ref]
    x_srcs = [x_ref, act1, act2]
    a_dsts = [act1, act2, out_ref]

    def load(k):
        l, b = divmod(k, _NB)
        s = k % _NBUF
        cp = pltpu.make_async_copy(
            w_hbms[l].at[pl.ds(b * _R, _R), :], bufs.at[s], load_sems.at[s])
        cp.start()
        return cp

    def store(k):
        l, b = divmod(k, _NB)
        s = k % _NBUF
        cp = pltpu.make_async_copy(
            bufs.at[s], o_hbms[l].at[pl.ds(b * _R, _R), :], store_sems.at[s])
        cp.start()
        return cp

    loads, stores = {}, {}
    for k in range(min(_D, _M)):
        loads[k] = load(k)

    for k in range(_M):
        l, b = divmod(k, _NB)
        s = k % _NBUF
        kd = k + _D
        if kd < _M:
            if kd >= _NBUF:
                stores[kd - _NBUF].wait()
            loads[kd] = load(kd)
        loads[k].wait()

        x = x_srcs[l][...]
        w = bufs[s]
        y = jnp.sum(w * x, axis=1)
        a = jnp.maximum(y + b_refs[l][pl.ds(b * _R, _R)], 0.0)
        a_dsts[l][0:1, pl.ds(b * _R, _R)] = a[None, :]

        if b == 0:
            a0 = jnp.sum(jnp.where(jax.lax.iota(jnp.int32, _R) == 0, a, 0.0))
            patch = (mw_ref[0, 0] * x_srcs[l][0, 0]
                     + mw_ref[0, 1] * bufs[s][0, 0]
                     + mw_ref[0, 2] * a0
                     + mb_ref[0])
            rows = jax.lax.broadcasted_iota(jnp.int32, (8, 128), 0)
            cols = jax.lax.broadcasted_iota(jnp.int32, (8, 128), 1)
            tile = bufs[s, 0:8, 0:128]
            bufs[s, 0:8, 0:128] = jnp.where(
                (rows == 0) & (cols == 0), patch, tile)

        stores[k] = store(k)

    for k in range(max(0, _M - _NBUF), _M):
        stores[k].wait()


def kernel(x, W1, b1, W2, b2, W3, b3, meta_W, meta_b):
    f32 = jnp.float32
    nw1, nw2, nw3, out = pl.pallas_call(
        _body,
        in_specs=[
            pl.BlockSpec(memory_space=pltpu.VMEM),
            pl.BlockSpec(memory_space=pltpu.VMEM),
            pl.BlockSpec(memory_space=pltpu.VMEM),
            pl.BlockSpec(memory_space=pltpu.VMEM),
            pl.BlockSpec(memory_space=pltpu.VMEM),
            pl.BlockSpec(memory_space=pltpu.VMEM),
            pl.BlockSpec(memory_space=pl.ANY),
            pl.BlockSpec(memory_space=pl.ANY),
            pl.BlockSpec(memory_space=pl.ANY),
        ],
        out_specs=[
            pl.BlockSpec(memory_space=pl.ANY),
            pl.BlockSpec(memory_space=pl.ANY),
            pl.BlockSpec(memory_space=pl.ANY),
            pl.BlockSpec(memory_space=pltpu.VMEM),
        ],
        out_shape=[
            jax.ShapeDtypeStruct((_N, _N), f32),
            jax.ShapeDtypeStruct((_N, _N), f32),
            jax.ShapeDtypeStruct((_N, _N), f32),
            jax.ShapeDtypeStruct((1, _N), f32),
        ],
        scratch_shapes=[
            pltpu.VMEM((_NBUF, _R, _N), f32),
            pltpu.VMEM((1, _N), f32),
            pltpu.VMEM((1, _N), f32),
            pltpu.SemaphoreType.DMA((_NBUF,)),
            pltpu.SemaphoreType.DMA((_NBUF,)),
        ],
    )(x, b1, b2, b3, meta_W, meta_b, W1, W2, W3)
    return out, nw1, nw2, nw3
